# Initial kernel scaffold; baseline (speedup 1.0000x reference)
import jax
import jax.numpy as jnp
from jax.experimental import pallas as pl


def kernel(x, index):
    raise NotImplementedError("write your pallas kernel here")


# speedup vs baseline: 4.4904x; 4.4904x over previous
import functools
import math

import jax
import jax.numpy as jnp
from jax import lax
from jax.experimental import pallas as pl
from jax.experimental.pallas import tpu as pltpu
from jax.experimental.pallas import tpu_sc as plsc

N_NODES = 10000
N_PAD = 10240
N_EDGES = 320000
D_FEAT = 128
SCALE = 1.0 / math.sqrt(32.0)

NC = 2
NS = 16
NW = NC * NS
E_PER_W = N_EDGES // NW
CHUNK = 80
NCHUNK = E_PER_W // CHUNK
ROWS_PER_TILE = N_PAD // NS


def _sc_scatter_partials(x, idx3):
    mesh = plsc.VectorSubcoreMesh(core_axis_name="c", subcore_axis_name="s")

    @functools.partial(
        pl.kernel,
        mesh=mesh,
        out_type=jax.ShapeDtypeStruct((NC, N_PAD, D_FEAT), jnp.float32),
        scratch_types=[
            pltpu.VMEM_SHARED((N_PAD, D_FEAT), jnp.float32),
            pltpu.VMEM((CHUNK, D_FEAT), jnp.float32),
            pltpu.VMEM((NCHUNK, CHUNK), jnp.int32),
            pltpu.VMEM((CHUNK, D_FEAT), jnp.float32),
        ],
    )
    def body(x_hbm, idx_hbm, out_hbm, acc, xbuf, idxb, zbuf):
        core = lax.axis_index("c")
        sub = lax.axis_index("s")
        wid = core * NS + sub

        zeros16 = jnp.zeros((16,), jnp.float32)

        def zrow(r, _):
            for c in range(D_FEAT // 16):
                zbuf[r, pl.ds(c * 16, 16)] = zeros16
            return 0

        lax.fori_loop(0, CHUNK, zrow, 0)
        acc_base = sub * ROWS_PER_TILE
        for b in range(ROWS_PER_TILE // CHUNK):
            pltpu.sync_copy(zbuf, acc.at[pl.ds(acc_base + b * CHUNK, CHUNK), :])
        plsc.subcore_barrier()

        pltpu.sync_copy(idx_hbm.at[wid], idxb)

        ebase = wid * E_PER_W

        def chunk_body(j, _):
            pltpu.sync_copy(x_hbm.at[pl.ds(ebase + j * CHUNK, CHUNK), :], xbuf)
            pltpu.sync_copy(xbuf, acc.at[idxb.at[j]], add=True)
            return 0

        lax.fori_loop(0, NCHUNK, chunk_body, 0)
        plsc.subcore_barrier()

        pltpu.sync_copy(
            acc.at[pl.ds(acc_base, ROWS_PER_TILE), :],
            out_hbm.at[core, pl.ds(acc_base, ROWS_PER_TILE), :],
        )

    return body(x, idx3)


def _combine(p_ref, o_ref):
    o_ref[...] = (p_ref[0] + p_ref[1]) * SCALE


def _tc_combine(partials):
    rows = 400
    return pl.pallas_call(
        _combine,
        grid=(N_NODES // rows,),
        in_specs=[pl.BlockSpec((NC, rows, D_FEAT), lambda i: (0, i, 0))],
        out_specs=pl.BlockSpec((rows, D_FEAT), lambda i: (i, 0)),
        out_shape=jax.ShapeDtypeStruct((N_NODES, D_FEAT), jnp.float32),
    )(partials)


def kernel(x, index):
    idx3 = index.astype(jnp.int32).reshape(NW, NCHUNK, CHUNK)
    partials = _sc_scatter_partials(x, idx3)
    return _tc_combine(partials)
